# baseline (speedup 1.0000x reference)
import functools

import jax
import jax.numpy as jnp
from jax import lax
from jax.experimental import pallas as pl
from jax.experimental.pallas import tpu as pltpu
from jax.experimental.pallas import tpu_sc as plsc

N = 10000
E = 200000
H = 2
D = 64
HID = 128
NL = 3
NT = 3

N1 = 10112
NC = 2
NS = 16
K = 128
EPT = 6400
NCH = EPT // K
EP = NC * NS * EPT
RPT = N1 // NS
RL = 80
KH = 64
NCHH = EPT // KH
BM = 1264
GRID = N1 // BM

_f32 = jnp.float32
_i32 = jnp.int32


def _lrelu(x):
    return jnp.where(x >= 0, x, x * 0.2)


_MESH = plsc.VectorSubcoreMesh(core_axis_name="c", subcore_axis_name="s")
_SC_PARAMS = pltpu.CompilerParams(use_tc_tiling_on_sc=False,
                                  needs_layout_passes=False)


def _zero_accum(ext, U, row0):
    rows = ext.shape[0]
    zv = jnp.zeros((16,), _f32)

    def zrow(r, _):
        for c in range(RL // 16):
            ext[r, pl.ds(c * 16, 16)] = zv
        return 0

    lax.fori_loop(0, rows, zrow, 0)
    for b in range(RPT // rows):
        pltpu.sync_copy(ext, U.at[pl.ds(row0 + b * rows, rows)])
    rem = RPT % rows
    if rem:
        pltpu.sync_copy(ext.at[pl.ds(0, rem)],
                        U.at[pl.ds(row0 + (RPT // rows) * rows, rem)])


def _writeout(U, msg_out, den_out, row0):
    pltpu.sync_copy(U.at[pl.ds(row0, RPT), pl.ds(0, D)],
                    msg_out.at[pl.ds(row0, RPT)])
    pltpu.sync_copy(U.at[pl.ds(row0, RPT), pl.ds(D, 16)],
                    den_out.at[pl.ds(row0, RPT)])


def _gat_sc_body(xw00, xw01, xw10, xw11, xw20, xw21,
                 sc00, sc01, sc10, sc11, sc20, sc21,
                 e0, e1, e2, msg_out, den_out,
                 tbl, sidx, didx, didx_s0, didx_s1, gbuf0, gbuf1,
                 ext0, ext1, U, sem0, sem1, semc0, semc1):
    cid = lax.axis_index("c")
    sid = lax.axis_index("s")
    row0 = sid * RPT
    ebase = cid * (NS * EPT) + sid * EPT

    for t, (xwp, scp, e_h) in enumerate(
            (((xw00, xw01), (sc00, sc01), e0),
             ((xw10, xw11), (sc10, sc11), e1),
             ((xw20, xw21), (sc20, sc21), e2))):
        pltpu.sync_copy(e_h.at[0, pl.ds(ebase, EPT)], sidx)
        pltpu.sync_copy(e_h.at[1, pl.ds(ebase, EPT)], didx)
        for hh in range(H):
            xw_h = xwp[hh]
            pltpu.sync_copy(scp[hh], tbl)
            _zero_accum(ext0, U, row0)
            plsc.subcore_barrier()

            def gidx(ch):
                return sidx.at[pl.ds(ch * K, K)]

            def work(ch, gbuf, ext, didx_s, semc):
                @pl.when(ch >= 2)
                def _():
                    pltpu.make_async_copy(ext, U.at[didx_s], semc).wait()

                o0 = ch * K

                @plsc.parallel_loop(0, K // 16, unroll=1)
                def _(g):
                    o = o0 + g * 16
                    sv = sidx[pl.ds(o, 16)] * 2
                    dvr = didx[pl.ds(o, 16)]
                    didx_s[pl.ds(g * 16, 16)] = dvr
                    dv = dvr * 2
                    aa = (plsc.load_gather(tbl, [sv])
                          + plsc.load_gather(tbl, [dv + 1]))
                    wv = jnp.exp(_lrelu(aa))
                    b16 = g * 16

                    @plsc.parallel_loop(0, 16, unroll=8)
                    def _(l):
                        er = b16 + l
                        ws = jnp.take(wv, jnp.full((16,), 0, _i32) + l)
                        for c in range(4):
                            ext[er, pl.ds(c * 16, 16)] = (
                                gbuf[er, pl.ds(c * 16, 16)] * ws)
                        ext[er, pl.ds(D, 16)] = ws

                pltpu.async_copy(ext, U.at[didx_s], semc, add=True)

            pltpu.async_copy(xw_h.at[gidx(0)], gbuf0, sem0)

            def pair(i, _):
                c0 = i * 2
                pltpu.make_async_copy(xw_h.at[gidx(c0)], gbuf0, sem0).wait()
                pltpu.async_copy(xw_h.at[gidx(c0 + 1)], gbuf1, sem1)
                work(c0, gbuf0, ext0, didx_s0, semc0)
                pltpu.make_async_copy(xw_h.at[gidx(c0 + 1)], gbuf1,
                                      sem1).wait()

                @pl.when(c0 + 2 < NCH)
                def _():
                    pltpu.async_copy(xw_h.at[gidx(c0 + 2)], gbuf0, sem0)

                work(c0 + 1, gbuf1, ext1, didx_s1, semc1)
                return 0

            lax.fori_loop(0, NCH // 2, pair, 0)
            pltpu.make_async_copy(ext0, U.at[didx_s0], semc0).wait()
            pltpu.make_async_copy(ext1, U.at[didx_s1], semc1).wait()
            plsc.subcore_barrier()
            _writeout(U, msg_out.at[t, hh, cid], den_out.at[t, hh, cid], row0)
            plsc.subcore_barrier()


def _hgt_sc_body(q0, q1, ke00, ke01, ke10, ke11, ke20, ke21,
                 ve00, ve01, ve10, ve11, ve20, ve21, pv_h, e0, e1, e2,
                 msg_out, den_out, pvb, sidx, didx, didx_s0, didx_s1,
                 qbuf0, kbuf0, vbuf0, qbuf1, kbuf1, vbuf1, ext0, ext1,
                 U, sem0, sem1, semc0, semc1):
    cid = lax.axis_index("c")
    sid = lax.axis_index("s")
    row0 = sid * RPT
    ebase = cid * (NS * EPT) + sid * EPT

    pltpu.sync_copy(pv_h, pvb)
    pv = pvb[...]
    qs = (q0, q1)
    kes = ((ke00, ke01), (ke10, ke11), (ke20, ke21))
    ves = ((ve00, ve01), (ve10, ve11), (ve20, ve21))
    slots = ((qbuf0, kbuf0, vbuf0, sem0, ext0, didx_s0, semc0),
             (qbuf1, kbuf1, vbuf1, sem1, ext1, didx_s1, semc1))

    for hh in range(H):
        q_h = qs[hh]
        _zero_accum(ext0, U, row0)
        plsc.subcore_barrier()
        for t in range(NT):
            ke_h = kes[t][hh]
            ve_h = ves[t][hh]
            e_h = (e0, e1, e2)[t]
            ps = jnp.take(pv, jnp.full((16,), 2 * t + hh, _i32))
            pltpu.sync_copy(e_h.at[0, pl.ds(ebase, EPT)], sidx)
            pltpu.sync_copy(e_h.at[1, pl.ds(ebase, EPT)], didx)

            def issue(ch, slot):
                qb, kb, vb, sem = slots[slot][:4]
                si = sidx.at[pl.ds(ch * KH, KH)]
                di = didx.at[pl.ds(ch * KH, KH)]
                pltpu.async_copy(q_h.at[di], qb, sem)
                pltpu.async_copy(ke_h.at[si], kb, sem)
                pltpu.async_copy(ve_h.at[si], vb, sem)

            def drain(ch, slot):
                qb, kb, vb, sem = slots[slot][:4]
                si = sidx.at[pl.ds(ch * KH, KH)]
                di = didx.at[pl.ds(ch * KH, KH)]
                pltpu.make_async_copy(q_h.at[di], qb, sem).wait()
                pltpu.make_async_copy(ke_h.at[si], kb, sem).wait()
                pltpu.make_async_copy(ve_h.at[si], vb, sem).wait()

            def work(ch, slot, first_type):
                qb, kb, vb, _, ext, didx_s, semc = slots[slot]
                if first_type:
                    @pl.when(ch >= 2)
                    def _():
                        pltpu.make_async_copy(ext, U.at[didx_s], semc).wait()
                else:
                    pltpu.make_async_copy(ext, U.at[didx_s], semc).wait()

                o0 = ch * KH

                def group(g, _):
                    o = o0 + g * 16
                    didx_s[pl.ds(g * 16, 16)] = didx[pl.ds(o, 16)]
                    b16 = g * 16

                    @plsc.parallel_loop(0, 16, unroll=4)
                    def _(l):
                        er = b16 + l
                        acc = qb[er, pl.ds(0, 16)] * kb[er, pl.ds(0, 16)]
                        for c in range(1, 4):
                            acc = acc + (qb[er, pl.ds(c * 16, 16)]
                                         * kb[er, pl.ds(c * 16, 16)])
                        av = jnp.sum(acc)
                        ws = jnp.exp(jnp.broadcast_to(av, (16,)) * ps)
                        for c in range(4):
                            ext[er, pl.ds(c * 16, 16)] = (
                                vb[er, pl.ds(c * 16, 16)] * ws)
                        ext[er, pl.ds(D, 16)] = ws

                    return 0

                lax.fori_loop(0, KH // 16, group, 0)
                pltpu.async_copy(ext, U.at[didx_s], semc, add=True)

            issue(0, 0)

            def pair(i, _):
                c0 = i * 2
                drain(c0, 0)
                issue(c0 + 1, 1)
                work(c0, 0, t == 0)
                drain(c0 + 1, 1)

                @pl.when(c0 + 2 < NCHH)
                def _():
                    issue(c0 + 2, 0)

                work(c0 + 1, 1, t == 0)
                return 0

            lax.fori_loop(0, NCHH // 2, pair, 0)
        pltpu.make_async_copy(ext0, U.at[didx_s0], semc0).wait()
        pltpu.make_async_copy(ext1, U.at[didx_s1], semc1).wait()
        plsc.subcore_barrier()
        _writeout(U, msg_out.at[hh, cid], den_out.at[hh, cid], row0)
        plsc.subcore_barrier()


_gat_sc = functools.partial(
    pl.kernel, _gat_sc_body, mesh=_MESH, compiler_params=_SC_PARAMS,
    out_type=[jax.ShapeDtypeStruct((NT, H, NC, N1, D), _f32),
              jax.ShapeDtypeStruct((NT, H, NC, N1, 16), _f32)],
    scratch_types=[pltpu.VMEM((N1 * 2,), _f32),
                   pltpu.VMEM((EPT,), _i32),
                   pltpu.VMEM((EPT,), _i32),
                   pltpu.VMEM((K,), _i32),
                   pltpu.VMEM((K,), _i32),
                   pltpu.VMEM((K, D), _f32),
                   pltpu.VMEM((K, D), _f32),
                   pltpu.VMEM((K, RL), _f32),
                   pltpu.VMEM((K, RL), _f32),
                   pltpu.VMEM_SHARED((N1, RL), _f32),
                   pltpu.SemaphoreType.DMA,
                   pltpu.SemaphoreType.DMA,
                   pltpu.SemaphoreType.DMA,
                   pltpu.SemaphoreType.DMA],
)()

_hgt_sc = functools.partial(
    pl.kernel, _hgt_sc_body, mesh=_MESH, compiler_params=_SC_PARAMS,
    out_type=[jax.ShapeDtypeStruct((H, NC, N1, D), _f32),
              jax.ShapeDtypeStruct((H, NC, N1, 16), _f32)],
    scratch_types=[pltpu.VMEM((16,), _f32),
                   pltpu.VMEM((EPT,), _i32),
                   pltpu.VMEM((EPT,), _i32),
                   pltpu.VMEM((KH,), _i32),
                   pltpu.VMEM((KH,), _i32),
                   pltpu.VMEM((KH, D), _f32),
                   pltpu.VMEM((KH, D), _f32),
                   pltpu.VMEM((KH, D), _f32),
                   pltpu.VMEM((KH, D), _f32),
                   pltpu.VMEM((KH, D), _f32),
                   pltpu.VMEM((KH, D), _f32),
                   pltpu.VMEM((KH, RL), _f32),
                   pltpu.VMEM((KH, RL), _f32),
                   pltpu.VMEM_SHARED((N1, RL), _f32),
                   pltpu.SemaphoreType.DMA,
                   pltpu.SemaphoreType.DMA,
                   pltpu.SemaphoreType.DMA,
                   pltpu.SemaphoreType.DMA],
)()


def _tca_body(h_ref, w3_ref, am_ref, xw00, xw01, xw10, xw11, xw20, xw21,
              s00, s01, s10, s11, s20, s21):
    hb = h_ref[...]
    xwps = ((xw00, xw01), (xw10, xw11), (xw20, xw21))
    scs = ((s00, s01), (s10, s11), (s20, s21))
    for t in range(NT):
        xw = jnp.dot(hb, w3_ref[t], preferred_element_type=_f32)
        xwps[t][0][...] = xw[:, :D]
        xwps[t][1][...] = xw[:, D:]
        sc = jnp.dot(xw, am_ref[t], preferred_element_type=_f32)
        scs[t][0][...] = sc[:, 0:2]
        scs[t][1][...] = sc[:, 2:4]


def _tc_a(h, W3, att_mat):
    return pl.pallas_call(
        _tca_body,
        grid=(GRID,),
        in_specs=[pl.BlockSpec((BM, HID), lambda i: (i, 0)),
                  pl.BlockSpec((NT, HID, HID), lambda i: (0, 0, 0)),
                  pl.BlockSpec((NT, HID, 4), lambda i: (0, 0, 0))],
        out_specs=[pl.BlockSpec((BM, D), lambda i: (i, 0))] * 6
        + [pl.BlockSpec((BM, 2), lambda i: (i, 0))] * 6,
        out_shape=[jax.ShapeDtypeStruct((N1, D), _f32)] * 6
        + [jax.ShapeDtypeStruct((N1, 2), _f32)] * 6,
    )(h, W3, att_mat)


def _tcb_body(msg_ref, den_ref, s00, s01, s10, s11, s20, s21,
              xw00, xw01, xw10, xw11, xw20, xw21,
              bg_ref, hsum_ref, st_ref):
    i = pl.program_id(0)
    xwps = ((xw00, xw01), (xw10, xw11), (xw20, xw21))
    scs = ((s00, s01), (s10, s11), (s20, s21))
    hs = None
    for t in range(NT):
        cols = []
        for hh in range(H):
            sc = scs[t][hh][...]
            es = jnp.exp(_lrelu(sc[:, 0] + sc[:, 1]))
            xw = xwps[t][hh][...]
            num = msg_ref[t, hh, 0] + msg_ref[t, hh, 1] + es[:, None] * xw
            den = (den_ref[t, hh, 0, :, 0] + den_ref[t, hh, 1, :, 0]
                   + es + 1e-16)
            cols.append(num / den[:, None])
        out = jnp.concatenate(cols, axis=1)
        hs = out if hs is None else hs + out
    hs = hs + (bg_ref[0] + bg_ref[1] + bg_ref[2])[None, :]
    hsum_ref[...] = hs
    rows = i * BM + lax.broadcasted_iota(_i32, (BM, 1), 0)
    hm = jnp.where(rows < N, hs, 0.0)
    st = jnp.stack([jnp.sum(hm, axis=0), jnp.sum(hm * hm, axis=0)])

    @pl.when(i == 0)
    def _():
        st_ref[...] = jnp.zeros((2, HID), _f32)

    st_ref[...] += st


def _tc_b(msg, den, scs, xws, bg):
    return pl.pallas_call(
        _tcb_body,
        grid=(GRID,),
        in_specs=[pl.BlockSpec((NT, H, NC, BM, D), lambda i: (0, 0, 0, i, 0)),
                  pl.BlockSpec((NT, H, NC, BM, 16), lambda i: (0, 0, 0, i, 0))]
        + [pl.BlockSpec((BM, 2), lambda i: (i, 0))] * 6
        + [pl.BlockSpec((BM, D), lambda i: (i, 0))] * 6
        + [pl.BlockSpec((NT, HID), lambda i: (0, 0))],
        out_specs=[pl.BlockSpec((BM, HID), lambda i: (i, 0)),
                   pl.BlockSpec((2, HID), lambda i: (0, 0))],
        out_shape=[jax.ShapeDtypeStruct((N1, HID), _f32),
                   jax.ShapeDtypeStruct((2, HID), _f32)],
    )(msg, den, *scs, *xws, bg)


def _tcc_body(hsum_ref, st_ref, g_ref, b_ref, hprev_ref, x0_ref, wp_ref,
              bp_ref, h_ref, *, first):
    i = pl.program_id(0)
    mean = st_ref[0] / float(N)
    var = st_ref[1] / float(N) - mean * mean
    inv = lax.rsqrt(var + 1e-5)
    xb = (hsum_ref[...] - mean[None, :]) * inv[None, :] * g_ref[...][None, :] \
        + b_ref[...][None, :]
    if first:
        hv = _lrelu(xb)
    else:
        xb = xb + jnp.dot(x0_ref[...], wp_ref[...],
                          preferred_element_type=_f32) + bp_ref[...][None, :]
        hv = _lrelu(hprev_ref[...] + xb)
    rows = i * BM + lax.broadcasted_iota(_i32, (BM, 1), 0)
    h_ref[...] = jnp.where(rows < N, hv, 0.0)


def _tc_c(hsum, st, g, b, hprev, x0p, wp, bp, first):
    return pl.pallas_call(
        functools.partial(_tcc_body, first=first),
        grid=(GRID,),
        in_specs=[pl.BlockSpec((BM, HID), lambda i: (i, 0)),
                  pl.BlockSpec((2, HID), lambda i: (0, 0)),
                  pl.BlockSpec((HID,), lambda i: (0,)),
                  pl.BlockSpec((HID,), lambda i: (0,)),
                  pl.BlockSpec((BM, HID), lambda i: (i, 0)),
                  pl.BlockSpec((BM, HID), lambda i: (i, 0)),
                  pl.BlockSpec((HID, HID), lambda i: (0, 0)),
                  pl.BlockSpec((HID,), lambda i: (0,))],
        out_specs=pl.BlockSpec((BM, HID), lambda i: (i, 0)),
        out_shape=jax.ShapeDtypeStruct((N1, HID), _f32),
    )(hsum, st, g, b, hprev, x0p, wp, bp)


def _tcd_body(h_ref, wk_ref, bk_ref, wq_ref, bq_ref, wv_ref, bv_ref,
              ar_ref, mr_ref, q0, q1, ke00, ke01, ke10, ke11, ke20, ke21,
              ve00, ve01, ve10, ve11, ve20, ve21):
    hb = h_ref[...]
    kb = jnp.dot(hb, wk_ref[...], preferred_element_type=_f32) \
        + bk_ref[...][None, :]
    qb = jnp.dot(hb, wq_ref[...], preferred_element_type=_f32) \
        + bq_ref[...][None, :]
    vb = jnp.dot(hb, wv_ref[...], preferred_element_type=_f32) \
        + bv_ref[...][None, :]
    q0[...] = qb[:, :D]
    q1[...] = qb[:, D:]
    kes = ((ke00, ke01), (ke10, ke11), (ke20, ke21))
    ves = ((ve00, ve01), (ve10, ve11), (ve20, ve21))
    for t in range(NT):
        for hh in range(H):
            kh = kb[:, hh * D:(hh + 1) * D]
            vh = vb[:, hh * D:(hh + 1) * D]
            kes[t][hh][...] = jnp.dot(kh, ar_ref[t, hh],
                                      preferred_element_type=_f32)
            ves[t][hh][...] = jnp.dot(vh, mr_ref[t, hh],
                                      preferred_element_type=_f32)


def _tc_d(h, Wk, bk, Wq, bq, Wv, bv, a_rel, m_rel):
    return pl.pallas_call(
        _tcd_body,
        grid=(GRID,),
        in_specs=[pl.BlockSpec((BM, HID), lambda i: (i, 0)),
                  pl.BlockSpec((HID, HID), lambda i: (0, 0)),
                  pl.BlockSpec((HID,), lambda i: (0,)),
                  pl.BlockSpec((HID, HID), lambda i: (0, 0)),
                  pl.BlockSpec((HID,), lambda i: (0,)),
                  pl.BlockSpec((HID, HID), lambda i: (0, 0)),
                  pl.BlockSpec((HID,), lambda i: (0,)),
                  pl.BlockSpec((NT, H, D, D), lambda i: (0, 0, 0, 0)),
                  pl.BlockSpec((NT, H, D, D), lambda i: (0, 0, 0, 0))],
        out_specs=[pl.BlockSpec((BM, D), lambda i: (i, 0))] * 14,
        out_shape=[jax.ShapeDtypeStruct((N1, D), _f32)] * 14,
    )(h, Wk, bk, Wq, bq, Wv, bv, a_rel, m_rel)


def _tce_body(msg_ref, den_ref, h_ref, wo_ref, bo_ref, sk_ref,
              hn_ref, st_ref):
    i = pl.program_id(0)
    cols = []
    for hh in range(H):
        u = msg_ref[hh, 0] + msg_ref[hh, 1]
        den = den_ref[hh, 0, :, 0] + den_ref[hh, 1, :, 0] + 1e-16
        cols.append(u / den[:, None])
    msg = jnp.concatenate(cols, axis=1)
    ge = 0.5 * msg * (1.0 + lax.erf(msg * (2.0 ** -0.5)))
    o2 = jnp.dot(ge, wo_ref[...], preferred_element_type=_f32) \
        + bo_ref[...][None, :]
    s = 1.0 / (1.0 + jnp.exp(-sk_ref[0, 0]))
    hn = s * o2 + (1.0 - s) * h_ref[...]
    hn_ref[...] = hn
    rows = i * BM + lax.broadcasted_iota(_i32, (BM, 1), 0)
    hm = jnp.where(rows < N, hn, 0.0)
    st = jnp.stack([jnp.sum(hm, axis=0), jnp.sum(hm * hm, axis=0)])

    @pl.when(i == 0)
    def _():
        st_ref[...] = jnp.zeros((2, HID), _f32)

    st_ref[...] += st


def _tc_e(msg, den, h, Wo, bo, skip):
    return pl.pallas_call(
        _tce_body,
        grid=(GRID,),
        in_specs=[pl.BlockSpec((H, NC, BM, D), lambda i: (0, 0, i, 0)),
                  pl.BlockSpec((H, NC, BM, 16), lambda i: (0, 0, i, 0)),
                  pl.BlockSpec((BM, HID), lambda i: (i, 0)),
                  pl.BlockSpec((HID, HID), lambda i: (0, 0)),
                  pl.BlockSpec((HID,), lambda i: (0,)),
                  pl.BlockSpec((1, 1), lambda i: (0, 0))],
        out_specs=[pl.BlockSpec((BM, HID), lambda i: (i, 0)),
                   pl.BlockSpec((2, HID), lambda i: (0, 0))],
        out_shape=[jax.ShapeDtypeStruct((N1, HID), _f32),
                   jax.ShapeDtypeStruct((2, HID), _f32)],
    )(msg, den, h, Wo, bo, skip)


def _tcf_body(hn_ref, st_ref, g_ref, b_ref, h_ref, wl_ref, bl_ref, y_ref):
    mean = st_ref[0] / float(N)
    var = st_ref[1] / float(N) - mean * mean
    inv = lax.rsqrt(var + 1e-5)
    xb = (hn_ref[...] - mean[None, :]) * inv[None, :] * g_ref[...][None, :] \
        + b_ref[...][None, :]
    hf = _lrelu(h_ref[...] + xb)
    y_ref[...] = jnp.dot(hf, wl_ref[...], preferred_element_type=_f32) \
        + bl_ref[0, 0]


def _tc_f(hn, st, g, b, h, Wl, bl):
    return pl.pallas_call(
        _tcf_body,
        grid=(GRID,),
        in_specs=[pl.BlockSpec((BM, HID), lambda i: (i, 0)),
                  pl.BlockSpec((2, HID), lambda i: (0, 0)),
                  pl.BlockSpec((HID,), lambda i: (0,)),
                  pl.BlockSpec((HID,), lambda i: (0,)),
                  pl.BlockSpec((BM, HID), lambda i: (i, 0)),
                  pl.BlockSpec((HID, 1), lambda i: (0, 0)),
                  pl.BlockSpec((1, 1), lambda i: (0, 0))],
        out_specs=pl.BlockSpec((BM, 1), lambda i: (i, 0)),
        out_shape=jax.ShapeDtypeStruct((N1, 1), _f32),
    )(hn, st, g, b, h, Wl, bl)


def _prep_edges(ei):
    s = ei[0].astype(_i32)
    d = ei[1].astype(_i32)
    pad = EP - E
    s = jnp.concatenate([s, jnp.full((pad,), N, _i32)])
    d = jnp.concatenate([d, jnp.full((pad,), N, _i32)])
    return jnp.stack([s, d])


def _att_mats(att_src, att_dst):
    z = jnp.zeros(att_src.shape[:2] + (D,), _f32)
    c0 = jnp.concatenate([att_src[:, :, 0, :], z], axis=-1)
    c1 = jnp.concatenate([att_dst[:, :, 0, :], z], axis=-1)
    c2 = jnp.concatenate([z, att_src[:, :, 1, :]], axis=-1)
    c3 = jnp.concatenate([z, att_dst[:, :, 1, :]], axis=-1)
    return jnp.stack([c0, c1, c2, c3], axis=-1)


def kernel(x_cell, edge_index_line, edge_index_region, edge_index_diag,
           W_gat, att_src, att_dst, b_gat, bn_gamma, bn_beta, Wp, bp,
           Wk, bk, Wq, bq, Wv, bv, a_rel, m_rel, p_rel, Wo, bo, skip,
           gf_gamma, gf_beta, Wl, bl):
    e0 = _prep_edges(edge_index_line)
    e1 = _prep_edges(edge_index_region)
    e2 = _prep_edges(edge_index_diag)
    x0p = jnp.pad(x_cell, ((0, N1 - N), (0, 0)))
    att_mat = _att_mats(att_src, att_dst)
    pv = jnp.pad(p_rel.reshape(NT * H) * (float(D) ** -0.5), (0, 16 - NT * H))
    skip2 = skip.reshape(1, 1)
    bl2 = bl.reshape(1, 1)

    h = x0p
    for li in range(NL):
        outs = _tc_a(h, W_gat[li], att_mat[li])
        xws, scs = outs[:6], outs[6:]
        msg, den = _gat_sc(*xws, *(s.reshape(-1) for s in scs), e0, e1, e2)
        hsum, st = _tc_b(msg, den, scs, xws, b_gat[li])
        if li == 0:
            h = _tc_c(hsum, st, bn_gamma[0], bn_beta[0], h, x0p,
                      jnp.zeros((HID, HID), _f32), jnp.zeros((HID,), _f32),
                      True)
        else:
            h = _tc_c(hsum, st, bn_gamma[li], bn_beta[li], h, x0p,
                      Wp[li - 1], bp[li - 1], False)

    kqv = _tc_d(h, Wk, bk, Wq, bq, Wv, bv, a_rel, m_rel)
    msgh, denh = _hgt_sc(*kqv, pv, e0, e1, e2)
    hn, st2 = _tc_e(msgh, denh, h, Wo, bo, skip2)
    y = _tc_f(hn, st2, gf_gamma, gf_beta, h, Wl, bl2)
    return y[:N, 0]


# speedup vs baseline: 1.8611x; 1.8611x over previous
import functools

import jax
import jax.numpy as jnp
from jax import lax
from jax.experimental import pallas as pl
from jax.experimental.pallas import tpu as pltpu
from jax.experimental.pallas import tpu_sc as plsc

N = 10000
E = 200000
H = 2
D = 64
HID = 128
NL = 3
NT = 3

N1 = 10112
NC = 2
NS = 16
K = 128
EPT = 6400
NCH = EPT // K
EP = NC * NS * EPT
RPT = N1 // NS
RL = 80
KH = 64
NCHH = EPT // KH
BM = 1264
GRID = N1 // BM

_f32 = jnp.float32
_i32 = jnp.int32


def _lrelu(x):
    return jnp.where(x >= 0, x, x * 0.2)


_MESH = plsc.VectorSubcoreMesh(core_axis_name="c", subcore_axis_name="s")
_SC_PARAMS = pltpu.CompilerParams(use_tc_tiling_on_sc=False,
                                  needs_layout_passes=False)


def _zero_accum(ext, U, row0):
    rows = ext.shape[0]
    zv = jnp.zeros((16,), _f32)

    def zrow(r, _):
        for c in range(RL // 16):
            ext[r, pl.ds(c * 16, 16)] = zv
        return 0

    lax.fori_loop(0, rows, zrow, 0)
    for b in range(RPT // rows):
        pltpu.sync_copy(ext, U.at[pl.ds(row0 + b * rows, rows)])
    rem = RPT % rows
    if rem:
        pltpu.sync_copy(ext.at[pl.ds(0, rem)],
                        U.at[pl.ds(row0 + (RPT // rows) * rows, rem)])


def _writeout(U, msg_out, den_out, row0):
    pltpu.sync_copy(U.at[pl.ds(row0, RPT), pl.ds(0, D)],
                    msg_out.at[pl.ds(row0, RPT)])
    pltpu.sync_copy(U.at[pl.ds(row0, RPT), pl.ds(D, 16)],
                    den_out.at[pl.ds(row0, RPT)])


def _gat_sc_body(xw00, xw01, xw10, xw11, xw20, xw21,
                 sc00, sc01, sc10, sc11, sc20, sc21,
                 e0, e1, e2, msg_out, den_out,
                 tbl, sidx, didx, didx_s0, didx_s1, gbuf0, gbuf1,
                 ext0, ext1, U, sem0, sem1, semc0, semc1):
    cid = lax.axis_index("c")
    sid = lax.axis_index("s")
    row0 = sid * RPT
    ebase = cid * (NS * EPT) + sid * EPT

    for t, (xwp, scp, e_h) in enumerate(
            (((xw00, xw01), (sc00, sc01), e0),
             ((xw10, xw11), (sc10, sc11), e1),
             ((xw20, xw21), (sc20, sc21), e2))):
        pltpu.sync_copy(e_h.at[0, pl.ds(ebase, EPT)], sidx)
        pltpu.sync_copy(e_h.at[1, pl.ds(ebase, EPT)], didx)
        for hh in range(H):
            xw_h = xwp[hh]
            pltpu.sync_copy(scp[hh], tbl)
            _zero_accum(ext0, U, row0)
            plsc.subcore_barrier()

            def gidx(ch):
                return sidx.at[pl.ds(ch * K, K)]

            def work(ch, gbuf, ext, didx_s, semc):
                @pl.when(ch >= 2)
                def _():
                    pltpu.make_async_copy(ext, U.at[didx_s], semc).wait()

                o0 = ch * K

                @plsc.parallel_loop(0, K // 16, unroll=1)
                def _(g):
                    o = o0 + g * 16
                    sv = sidx[pl.ds(o, 16)] * 2
                    dvr = didx[pl.ds(o, 16)]
                    didx_s[pl.ds(g * 16, 16)] = dvr
                    dv = dvr * 2
                    aa = (plsc.load_gather(tbl, [sv])
                          + plsc.load_gather(tbl, [dv + 1]))
                    wv = jnp.exp(_lrelu(aa))
                    b16 = g * 16

                    @plsc.parallel_loop(0, 16, unroll=8)
                    def _(l):
                        er = b16 + l
                        ws = jnp.take(wv, jnp.full((16,), 0, _i32) + l)
                        for c in range(4):
                            ext[er, pl.ds(c * 16, 16)] = (
                                gbuf[er, pl.ds(c * 16, 16)] * ws)
                        ext[er, pl.ds(D, 16)] = ws

                pltpu.async_copy(ext, U.at[didx_s], semc, add=True)

            pltpu.async_copy(xw_h.at[gidx(0)], gbuf0, sem0)

            def pair(i, _):
                c0 = i * 2
                pltpu.make_async_copy(xw_h.at[gidx(c0)], gbuf0, sem0).wait()
                pltpu.async_copy(xw_h.at[gidx(c0 + 1)], gbuf1, sem1)
                work(c0, gbuf0, ext0, didx_s0, semc0)
                pltpu.make_async_copy(xw_h.at[gidx(c0 + 1)], gbuf1,
                                      sem1).wait()

                @pl.when(c0 + 2 < NCH)
                def _():
                    pltpu.async_copy(xw_h.at[gidx(c0 + 2)], gbuf0, sem0)

                work(c0 + 1, gbuf1, ext1, didx_s1, semc1)
                return 0

            lax.fori_loop(0, NCH // 2, pair, 0)
            pltpu.make_async_copy(ext0, U.at[didx_s0], semc0).wait()
            pltpu.make_async_copy(ext1, U.at[didx_s1], semc1).wait()
            plsc.subcore_barrier()
            _writeout(U, msg_out.at[t, hh, cid], den_out.at[t, hh, cid], row0)
            plsc.subcore_barrier()


def _hgt_sc_body(q0, q1, ke00, ke01, ke10, ke11, ke20, ke21,
                 ve00, ve01, ve10, ve11, ve20, ve21, pv_h, e0, e1, e2,
                 msg_out, den_out, pvb, sidx, didx, didx_s0, didx_s1,
                 qbuf0, kbuf0, vbuf0, qbuf1, kbuf1, vbuf1, ext0, ext1,
                 U, sem0, sem1, semc0, semc1):
    cid = lax.axis_index("c")
    sid = lax.axis_index("s")
    row0 = sid * RPT
    ebase = cid * (NS * EPT) + sid * EPT

    pltpu.sync_copy(pv_h, pvb)
    pv = pvb[...]
    qs = (q0, q1)
    kes = ((ke00, ke01), (ke10, ke11), (ke20, ke21))
    ves = ((ve00, ve01), (ve10, ve11), (ve20, ve21))
    slots = ((qbuf0, kbuf0, vbuf0, sem0, ext0, didx_s0, semc0),
             (qbuf1, kbuf1, vbuf1, sem1, ext1, didx_s1, semc1))

    for hh in range(H):
        q_h = qs[hh]
        _zero_accum(ext0, U, row0)
        plsc.subcore_barrier()
        for t in range(NT):
            ke_h = kes[t][hh]
            ve_h = ves[t][hh]
            e_h = (e0, e1, e2)[t]
            ps = jnp.take(pv, jnp.full((16,), 2 * t + hh, _i32))
            pltpu.sync_copy(e_h.at[0, pl.ds(ebase, EPT)], sidx)
            pltpu.sync_copy(e_h.at[1, pl.ds(ebase, EPT)], didx)

            def issue(ch, slot):
                qb, kb, vb, sem = slots[slot][:4]
                si = sidx.at[pl.ds(ch * KH, KH)]
                di = didx.at[pl.ds(ch * KH, KH)]
                pltpu.async_copy(q_h.at[di], qb, sem)
                pltpu.async_copy(ke_h.at[si], kb, sem)
                pltpu.async_copy(ve_h.at[si], vb, sem)

            def drain(ch, slot):
                qb, kb, vb, sem = slots[slot][:4]
                si = sidx.at[pl.ds(ch * KH, KH)]
                di = didx.at[pl.ds(ch * KH, KH)]
                pltpu.make_async_copy(q_h.at[di], qb, sem).wait()
                pltpu.make_async_copy(ke_h.at[si], kb, sem).wait()
                pltpu.make_async_copy(ve_h.at[si], vb, sem).wait()

            def work(ch, slot, first_type):
                qb, kb, vb, _, ext, didx_s, semc = slots[slot]
                if first_type:
                    @pl.when(ch >= 2)
                    def _():
                        pltpu.make_async_copy(ext, U.at[didx_s], semc).wait()
                else:
                    pltpu.make_async_copy(ext, U.at[didx_s], semc).wait()

                o0 = ch * KH

                def group(g, _):
                    o = o0 + g * 16
                    didx_s[pl.ds(g * 16, 16)] = didx[pl.ds(o, 16)]
                    b16 = g * 16

                    @plsc.parallel_loop(0, 16, unroll=4)
                    def _(l):
                        er = b16 + l
                        acc = qb[er, pl.ds(0, 16)] * kb[er, pl.ds(0, 16)]
                        for c in range(1, 4):
                            acc = acc + (qb[er, pl.ds(c * 16, 16)]
                                         * kb[er, pl.ds(c * 16, 16)])
                        av = jnp.sum(acc)
                        ws = jnp.exp(jnp.broadcast_to(av, (16,)) * ps)
                        for c in range(4):
                            ext[er, pl.ds(c * 16, 16)] = (
                                vb[er, pl.ds(c * 16, 16)] * ws)
                        ext[er, pl.ds(D, 16)] = ws

                    return 0

                lax.fori_loop(0, KH // 16, group, 0)
                pltpu.async_copy(ext, U.at[didx_s], semc, add=True)

            issue(0, 0)

            def pair(i, _):
                c0 = i * 2
                drain(c0, 0)
                issue(c0 + 1, 1)
                work(c0, 0, t == 0)
                drain(c0 + 1, 1)

                @pl.when(c0 + 2 < NCHH)
                def _():
                    issue(c0 + 2, 0)

                work(c0 + 1, 1, t == 0)
                return 0

            lax.fori_loop(0, NCHH // 2, pair, 0)
        pltpu.make_async_copy(ext0, U.at[didx_s0], semc0).wait()
        pltpu.make_async_copy(ext1, U.at[didx_s1], semc1).wait()
        plsc.subcore_barrier()
        _writeout(U, msg_out.at[hh, cid], den_out.at[hh, cid], row0)
        plsc.subcore_barrier()


_gat_sc = functools.partial(
    pl.kernel, _gat_sc_body, mesh=_MESH, compiler_params=_SC_PARAMS,
    out_type=[jax.ShapeDtypeStruct((NT, H, NC, N1, D), _f32),
              jax.ShapeDtypeStruct((NT, H, NC, N1, 16), _f32)],
    scratch_types=[pltpu.VMEM((N1 * 2,), _f32),
                   pltpu.VMEM((EPT,), _i32),
                   pltpu.VMEM((EPT,), _i32),
                   pltpu.VMEM((K,), _i32),
                   pltpu.VMEM((K,), _i32),
                   pltpu.VMEM((K, D), _f32),
                   pltpu.VMEM((K, D), _f32),
                   pltpu.VMEM((K, RL), _f32),
                   pltpu.VMEM((K, RL), _f32),
                   pltpu.VMEM_SHARED((N1, RL), _f32),
                   pltpu.SemaphoreType.DMA,
                   pltpu.SemaphoreType.DMA,
                   pltpu.SemaphoreType.DMA,
                   pltpu.SemaphoreType.DMA],
)()

_hgt_sc = functools.partial(
    pl.kernel, _hgt_sc_body, mesh=_MESH, compiler_params=_SC_PARAMS,
    out_type=[jax.ShapeDtypeStruct((H, NC, N1, D), _f32),
              jax.ShapeDtypeStruct((H, NC, N1, 16), _f32)],
    scratch_types=[pltpu.VMEM((16,), _f32),
                   pltpu.VMEM((EPT,), _i32),
                   pltpu.VMEM((EPT,), _i32),
                   pltpu.VMEM((KH,), _i32),
                   pltpu.VMEM((KH,), _i32),
                   pltpu.VMEM((KH, D), _f32),
                   pltpu.VMEM((KH, D), _f32),
                   pltpu.VMEM((KH, D), _f32),
                   pltpu.VMEM((KH, D), _f32),
                   pltpu.VMEM((KH, D), _f32),
                   pltpu.VMEM((KH, D), _f32),
                   pltpu.VMEM((KH, RL), _f32),
                   pltpu.VMEM((KH, RL), _f32),
                   pltpu.VMEM_SHARED((N1, RL), _f32),
                   pltpu.SemaphoreType.DMA,
                   pltpu.SemaphoreType.DMA,
                   pltpu.SemaphoreType.DMA,
                   pltpu.SemaphoreType.DMA],
)()


def _tca_body(h_ref, w3_ref, am_ref, xw00, xw01, xw10, xw11, xw20, xw21,
              s00, s01, s10, s11, s20, s21):
    hb = h_ref[...]
    xwps = ((xw00, xw01), (xw10, xw11), (xw20, xw21))
    scs = ((s00, s01), (s10, s11), (s20, s21))
    for t in range(NT):
        xw = jnp.dot(hb, w3_ref[t], preferred_element_type=_f32)
        xwps[t][0][...] = xw[:, :D]
        xwps[t][1][...] = xw[:, D:]
        sc = jnp.dot(xw, am_ref[t], preferred_element_type=_f32)
        scs[t][0][...] = sc[:, 0:2]
        scs[t][1][...] = sc[:, 2:4]


def _tc_a(h, W3, att_mat):
    return pl.pallas_call(
        _tca_body,
        grid=(GRID,),
        in_specs=[pl.BlockSpec((BM, HID), lambda i: (i, 0)),
                  pl.BlockSpec((NT, HID, HID), lambda i: (0, 0, 0)),
                  pl.BlockSpec((NT, HID, 4), lambda i: (0, 0, 0))],
        out_specs=[pl.BlockSpec((BM, D), lambda i: (i, 0))] * 6
        + [pl.BlockSpec((BM, 2), lambda i: (i, 0))] * 6,
        out_shape=[jax.ShapeDtypeStruct((N1, D), _f32)] * 6
        + [jax.ShapeDtypeStruct((N1, 2), _f32)] * 6,
    )(h, W3, att_mat)


def _tcb_body(msg_ref, den_ref, s00, s01, s10, s11, s20, s21,
              xw00, xw01, xw10, xw11, xw20, xw21,
              bg_ref, hsum_ref, st_ref):
    i = pl.program_id(0)
    xwps = ((xw00, xw01), (xw10, xw11), (xw20, xw21))
    scs = ((s00, s01), (s10, s11), (s20, s21))
    hs = None
    for t in range(NT):
        cols = []
        for hh in range(H):
            sc = scs[t][hh][...]
            es = jnp.exp(_lrelu(sc[:, 0] + sc[:, 1]))
            xw = xwps[t][hh][...]
            num = msg_ref[t, hh, 0] + msg_ref[t, hh, 1] + es[:, None] * xw
            den = (den_ref[t, hh, 0, :, 0] + den_ref[t, hh, 1, :, 0]
                   + es + 1e-16)
            cols.append(num / den[:, None])
        out = jnp.concatenate(cols, axis=1)
        hs = out if hs is None else hs + out
    hs = hs + (bg_ref[0] + bg_ref[1] + bg_ref[2])[None, :]
    hsum_ref[...] = hs
    rows = i * BM + lax.broadcasted_iota(_i32, (BM, 1), 0)
    hm = jnp.where(rows < N, hs, 0.0)
    st = jnp.stack([jnp.sum(hm, axis=0), jnp.sum(hm * hm, axis=0)])

    @pl.when(i == 0)
    def _():
        st_ref[...] = jnp.zeros((2, HID), _f32)

    st_ref[...] += st


def _tc_b(msg, den, scs, xws, bg):
    return pl.pallas_call(
        _tcb_body,
        grid=(GRID,),
        in_specs=[pl.BlockSpec((NT, H, NC, BM, D), lambda i: (0, 0, 0, i, 0)),
                  pl.BlockSpec((NT, H, NC, BM, 16), lambda i: (0, 0, 0, i, 0))]
        + [pl.BlockSpec((BM, 2), lambda i: (i, 0))] * 6
        + [pl.BlockSpec((BM, D), lambda i: (i, 0))] * 6
        + [pl.BlockSpec((NT, HID), lambda i: (0, 0))],
        out_specs=[pl.BlockSpec((BM, HID), lambda i: (i, 0)),
                   pl.BlockSpec((2, HID), lambda i: (0, 0))],
        out_shape=[jax.ShapeDtypeStruct((N1, HID), _f32),
                   jax.ShapeDtypeStruct((2, HID), _f32)],
    )(msg, den, *scs, *xws, bg)


def _tcc_body(hsum_ref, st_ref, g_ref, b_ref, hprev_ref, x0_ref, wp_ref,
              bp_ref, h_ref, *, first):
    i = pl.program_id(0)
    mean = st_ref[0] / float(N)
    var = st_ref[1] / float(N) - mean * mean
    inv = lax.rsqrt(var + 1e-5)
    xb = (hsum_ref[...] - mean[None, :]) * inv[None, :] * g_ref[...][None, :] \
        + b_ref[...][None, :]
    if first:
        hv = _lrelu(xb)
    else:
        xb = xb + jnp.dot(x0_ref[...], wp_ref[...],
                          preferred_element_type=_f32) + bp_ref[...][None, :]
        hv = _lrelu(hprev_ref[...] + xb)
    rows = i * BM + lax.broadcasted_iota(_i32, (BM, 1), 0)
    h_ref[...] = jnp.where(rows < N, hv, 0.0)


def _tc_c(hsum, st, g, b, hprev, x0p, wp, bp, first):
    return pl.pallas_call(
        functools.partial(_tcc_body, first=first),
        grid=(GRID,),
        in_specs=[pl.BlockSpec((BM, HID), lambda i: (i, 0)),
                  pl.BlockSpec((2, HID), lambda i: (0, 0)),
                  pl.BlockSpec((HID,), lambda i: (0,)),
                  pl.BlockSpec((HID,), lambda i: (0,)),
                  pl.BlockSpec((BM, HID), lambda i: (i, 0)),
                  pl.BlockSpec((BM, HID), lambda i: (i, 0)),
                  pl.BlockSpec((HID, HID), lambda i: (0, 0)),
                  pl.BlockSpec((HID,), lambda i: (0,))],
        out_specs=pl.BlockSpec((BM, HID), lambda i: (i, 0)),
        out_shape=jax.ShapeDtypeStruct((N1, HID), _f32),
    )(hsum, st, g, b, hprev, x0p, wp, bp)


def _tcd_body(h_ref, wk_ref, bk_ref, wq_ref, bq_ref, wv_ref, bv_ref,
              ar_ref, mr_ref, q0, q1, ke00, ke01, ke10, ke11, ke20, ke21,
              ve00, ve01, ve10, ve11, ve20, ve21):
    hb = h_ref[...]
    kb = jnp.dot(hb, wk_ref[...], preferred_element_type=_f32) \
        + bk_ref[...][None, :]
    qb = jnp.dot(hb, wq_ref[...], preferred_element_type=_f32) \
        + bq_ref[...][None, :]
    vb = jnp.dot(hb, wv_ref[...], preferred_element_type=_f32) \
        + bv_ref[...][None, :]
    q0[...] = qb[:, :D]
    q1[...] = qb[:, D:]
    kes = ((ke00, ke01), (ke10, ke11), (ke20, ke21))
    ves = ((ve00, ve01), (ve10, ve11), (ve20, ve21))
    for t in range(NT):
        for hh in range(H):
            kh = kb[:, hh * D:(hh + 1) * D]
            vh = vb[:, hh * D:(hh + 1) * D]
            kes[t][hh][...] = jnp.dot(kh, ar_ref[t, hh],
                                      preferred_element_type=_f32)
            ves[t][hh][...] = jnp.dot(vh, mr_ref[t, hh],
                                      preferred_element_type=_f32)


def _tc_d(h, Wk, bk, Wq, bq, Wv, bv, a_rel, m_rel):
    return pl.pallas_call(
        _tcd_body,
        grid=(GRID,),
        in_specs=[pl.BlockSpec((BM, HID), lambda i: (i, 0)),
                  pl.BlockSpec((HID, HID), lambda i: (0, 0)),
                  pl.BlockSpec((HID,), lambda i: (0,)),
                  pl.BlockSpec((HID, HID), lambda i: (0, 0)),
                  pl.BlockSpec((HID,), lambda i: (0,)),
                  pl.BlockSpec((HID, HID), lambda i: (0, 0)),
                  pl.BlockSpec((HID,), lambda i: (0,)),
                  pl.BlockSpec((NT, H, D, D), lambda i: (0, 0, 0, 0)),
                  pl.BlockSpec((NT, H, D, D), lambda i: (0, 0, 0, 0))],
        out_specs=[pl.BlockSpec((BM, D), lambda i: (i, 0))] * 14,
        out_shape=[jax.ShapeDtypeStruct((N1, D), _f32)] * 14,
    )(h, Wk, bk, Wq, bq, Wv, bv, a_rel, m_rel)


def _tce_body(msg_ref, den_ref, h_ref, wo_ref, bo_ref, sk_ref,
              hn_ref, st_ref):
    i = pl.program_id(0)
    cols = []
    for hh in range(H):
        u = msg_ref[hh, 0] + msg_ref[hh, 1]
        den = den_ref[hh, 0, :, 0] + den_ref[hh, 1, :, 0] + 1e-16
        cols.append(u / den[:, None])
    msg = jnp.concatenate(cols, axis=1)
    ge = 0.5 * msg * (1.0 + lax.erf(msg * (2.0 ** -0.5)))
    o2 = jnp.dot(ge, wo_ref[...], preferred_element_type=_f32) \
        + bo_ref[...][None, :]
    s = 1.0 / (1.0 + jnp.exp(-sk_ref[0, 0]))
    hn = s * o2 + (1.0 - s) * h_ref[...]
    hn_ref[...] = hn
    rows = i * BM + lax.broadcasted_iota(_i32, (BM, 1), 0)
    hm = jnp.where(rows < N, hn, 0.0)
    st = jnp.stack([jnp.sum(hm, axis=0), jnp.sum(hm * hm, axis=0)])

    @pl.when(i == 0)
    def _():
        st_ref[...] = jnp.zeros((2, HID), _f32)

    st_ref[...] += st


def _tc_e(msg, den, h, Wo, bo, skip):
    return pl.pallas_call(
        _tce_body,
        grid=(GRID,),
        in_specs=[pl.BlockSpec((H, NC, BM, D), lambda i: (0, 0, i, 0)),
                  pl.BlockSpec((H, NC, BM, 16), lambda i: (0, 0, i, 0)),
                  pl.BlockSpec((BM, HID), lambda i: (i, 0)),
                  pl.BlockSpec((HID, HID), lambda i: (0, 0)),
                  pl.BlockSpec((HID,), lambda i: (0,)),
                  pl.BlockSpec((1, 1), lambda i: (0, 0))],
        out_specs=[pl.BlockSpec((BM, HID), lambda i: (i, 0)),
                   pl.BlockSpec((2, HID), lambda i: (0, 0))],
        out_shape=[jax.ShapeDtypeStruct((N1, HID), _f32),
                   jax.ShapeDtypeStruct((2, HID), _f32)],
    )(msg, den, h, Wo, bo, skip)


def _tcf_body(hn_ref, st_ref, g_ref, b_ref, h_ref, wl_ref, bl_ref, y_ref):
    mean = st_ref[0] / float(N)
    var = st_ref[1] / float(N) - mean * mean
    inv = lax.rsqrt(var + 1e-5)
    xb = (hn_ref[...] - mean[None, :]) * inv[None, :] * g_ref[...][None, :] \
        + b_ref[...][None, :]
    hf = _lrelu(h_ref[...] + xb)
    y_ref[...] = jnp.dot(hf, wl_ref[...], preferred_element_type=_f32) \
        + bl_ref[0, 0]


def _tc_f(hn, st, g, b, h, Wl, bl):
    return pl.pallas_call(
        _tcf_body,
        grid=(GRID,),
        in_specs=[pl.BlockSpec((BM, HID), lambda i: (i, 0)),
                  pl.BlockSpec((2, HID), lambda i: (0, 0)),
                  pl.BlockSpec((HID,), lambda i: (0,)),
                  pl.BlockSpec((HID,), lambda i: (0,)),
                  pl.BlockSpec((BM, HID), lambda i: (i, 0)),
                  pl.BlockSpec((HID, 1), lambda i: (0, 0)),
                  pl.BlockSpec((1, 1), lambda i: (0, 0))],
        out_specs=pl.BlockSpec((BM, 1), lambda i: (i, 0)),
        out_shape=jax.ShapeDtypeStruct((N1, 1), _f32),
    )(hn, st, g, b, h, Wl, bl)


def _prep_edges(ei):
    s = ei[0].astype(_i32)
    d = ei[1].astype(_i32)
    pad = EP - E
    dummy = N + jnp.arange(pad, dtype=_i32) % (N1 - N)
    s = jnp.concatenate([s, dummy])
    d = jnp.concatenate([d, dummy])
    return jnp.stack([s, d])


def _att_mats(att_src, att_dst):
    z = jnp.zeros(att_src.shape[:2] + (D,), _f32)
    c0 = jnp.concatenate([att_src[:, :, 0, :], z], axis=-1)
    c1 = jnp.concatenate([att_dst[:, :, 0, :], z], axis=-1)
    c2 = jnp.concatenate([z, att_src[:, :, 1, :]], axis=-1)
    c3 = jnp.concatenate([z, att_dst[:, :, 1, :]], axis=-1)
    return jnp.stack([c0, c1, c2, c3], axis=-1)


def kernel(x_cell, edge_index_line, edge_index_region, edge_index_diag,
           W_gat, att_src, att_dst, b_gat, bn_gamma, bn_beta, Wp, bp,
           Wk, bk, Wq, bq, Wv, bv, a_rel, m_rel, p_rel, Wo, bo, skip,
           gf_gamma, gf_beta, Wl, bl):
    e0 = _prep_edges(edge_index_line)
    e1 = _prep_edges(edge_index_region)
    e2 = _prep_edges(edge_index_diag)
    x0p = jnp.pad(x_cell, ((0, N1 - N), (0, 0)))
    att_mat = _att_mats(att_src, att_dst)
    pv = jnp.pad(p_rel.reshape(NT * H) * (float(D) ** -0.5), (0, 16 - NT * H))
    skip2 = skip.reshape(1, 1)
    bl2 = bl.reshape(1, 1)

    h = x0p
    for li in range(NL):
        outs = _tc_a(h, W_gat[li], att_mat[li])
        xws, scs = outs[:6], outs[6:]
        msg, den = _gat_sc(*xws, *(s.reshape(-1) for s in scs), e0, e1, e2)
        hsum, st = _tc_b(msg, den, scs, xws, b_gat[li])
        if li == 0:
            h = _tc_c(hsum, st, bn_gamma[0], bn_beta[0], h, x0p,
                      jnp.zeros((HID, HID), _f32), jnp.zeros((HID,), _f32),
                      True)
        else:
            h = _tc_c(hsum, st, bn_gamma[li], bn_beta[li], h, x0p,
                      Wp[li - 1], bp[li - 1], False)

    kqv = _tc_d(h, Wk, bk, Wq, bq, Wv, bv, a_rel, m_rel)
    msgh, denh = _hgt_sc(*kqv, pv, e0, e1, e2)
    hn, st2 = _tc_e(msgh, denh, h, Wo, bo, skip2)
    y = _tc_f(hn, st2, gf_gamma, gf_beta, h, Wl, bl2)
    return y[:N, 0]
